# baseline (speedup 1.0000x reference)
import functools

import jax
import jax.numpy as jnp
from jax import lax
from jax.experimental import pallas as pl
from jax.experimental.pallas import tpu as pltpu
from jax.experimental.pallas import tpu_sc as plsc

D = 64
L = 16
NC, NS = 2, 16
NW = NC * NS
BLK = 128
EPS = 1e-5
UNROLL = 16
NEWTON = 1


def _ln_impl(idx, table2, gamma, beta):
  nblk = idx.shape[0]
  blk_per_w = nblk // NW

  mesh = plsc.VectorSubcoreMesh(
      core_axis_name="c", subcore_axis_name="s", num_cores=NC, num_subcores=NS
  )

  @functools.partial(
      pl.kernel,
      out_type=jax.ShapeDtypeStruct((nblk, BLK, D), jnp.float32),
      mesh=mesh,
      compiler_params=pltpu.CompilerParams(use_tc_tiling_on_sc=False),
      scratch_types=[
          pltpu.VMEM((blk_per_w, BLK), jnp.int32),
          pltpu.VMEM((2, BLK), jnp.int32),
          pltpu.VMEM((2, BLK), jnp.int32),
          pltpu.VMEM((2, BLK, 2 * D), jnp.float32),
          pltpu.VMEM((2, BLK, D), jnp.float32),
          pltpu.VMEM((D,), jnp.float32),
          pltpu.VMEM((D,), jnp.float32),
          pltpu.SemaphoreType.DMA,
          pltpu.SemaphoreType.DMA,
          pltpu.SemaphoreType.DMA,
          pltpu.SemaphoreType.DMA,
      ],
  )
  def k(idx_hbm, table_hbm, gamma_hbm, beta_hbm, out_hbm, idxall, idx2v,
        parv, ibuf, obuf, gamma_v, beta_v, gsem0, gsem1, osem0, osem1):
    wid = lax.axis_index("s") * NC + lax.axis_index("c")
    wbase = wid * blk_per_w

    pltpu.sync_copy(gamma_hbm, gamma_v)
    pltpu.sync_copy(beta_hbm, beta_v)
    pltpu.sync_copy(idx_hbm.at[pl.ds(wbase, blk_per_w)], idxall)

    gvec = [gamma_v[pl.ds(L * t, L)] for t in range(D // L)]
    bvec = [beta_v[pl.ds(L * t, L)] for t in range(D // L)]
    lane = lax.iota(jnp.int32, L)
    perms = [(lane + sh) & (L - 1) for sh in (8, 4, 2, 1)]

    def fire_gather(h, p, gsem):
      for j in range(BLK // L):
        v = idxall[h, pl.ds(L * j, L)]
        idx2v[p, pl.ds(L * j, L)] = lax.shift_right_logical(v, 1)
        parv[p, pl.ds(L * j, L)] = lax.shift_left(v & 1, 6)
      pltpu.async_copy(table_hbm.at[idx2v.at[p]], ibuf.at[p], gsem)

    def wait_gather(p, gsem):
      pltpu.make_async_copy(table_hbm.at[idx2v.at[p]], ibuf.at[p],
                            gsem).wait()

    def compute(p):
      def row16(rr, carry):
        pv = parv[p, pl.ds(rr * UNROLL, UNROLL)]
        for u in range(UNROLL):
          r = rr * UNROLL + u
          off = pv[u]
          x = [ibuf[p, r, pl.ds(off + L * t, L)] for t in range(D // L)]
          s = (x[0] + x[1]) + (x[2] + x[3])
          sq = (x[0] * x[0] + x[1] * x[1]) + (x[2] * x[2] + x[3] * x[3])
          for pm in perms:
            s = s + s.at[pm].get(mode="promise_in_bounds")
            sq = sq + sq.at[pm].get(mode="promise_in_bounds")
          mean_v = s * (1.0 / D)
          ex2 = sq * (1.0 / D)
          tv = ex2 - mean_v * mean_v + EPS
          seed = lax.bitcast_convert_type(tv, jnp.int32)
          seed = 0x5F3759DF - lax.shift_right_logical(seed, 1)
          g = lax.bitcast_convert_type(seed, jnp.float32)
          htv = 0.5 * tv
          for _ in range(NEWTON):
            g = g * (1.5 - htv * g * g)
          for t in range(D // L):
            obuf[p, r, pl.ds(L * t, L)] = (
                (x[t] - mean_v) * g * gvec[t] + bvec[t])
        return carry

      lax.fori_loop(0, BLK // UNROLL, row16, 0)

    def fire_out(h, p, osem):
      pltpu.async_copy(obuf.at[p], out_hbm.at[wbase + h], osem)

    def wait_out(p, osem):
      pltpu.make_async_copy(obuf.at[p], out_hbm.at[wbase], osem).wait()

    fire_gather(0, 0, gsem0)

    def pair_body(i, carry):
      ha = 2 * i
      fire_gather(ha + 1, 1, gsem1)
      wait_gather(0, gsem0)

      @pl.when(i > 0)
      def _():
        wait_out(0, osem0)

      compute(0)
      fire_out(ha, 0, osem0)

      @pl.when(i < blk_per_w // 2 - 1)
      def _():
        fire_gather(ha + 2, 0, gsem0)

      wait_gather(1, gsem1)

      @pl.when(i > 0)
      def _():
        wait_out(1, osem1)

      compute(1)
      fire_out(ha + 1, 1, osem1)
      return carry

    lax.fori_loop(0, blk_per_w // 2, pair_body, 0)
    wait_out(0, osem0)
    wait_out(1, osem1)

  return k(idx, table2, gamma, beta)


def kernel(job_id, table, gamma, beta):
  b, h = job_id.shape
  n = b * h
  idx = job_id.reshape(n // BLK, BLK).astype(jnp.int32)
  table2 = table.reshape(table.shape[0] // 2, 2 * D)
  out = _ln_impl(idx, table2, gamma, beta)
  return out.reshape(b, h, D)


# speedup vs baseline: 1.2658x; 1.2658x over previous
import functools

import jax
import jax.numpy as jnp
from jax import lax
from jax.experimental import pallas as pl
from jax.experimental.pallas import tpu as pltpu
from jax.experimental.pallas import tpu_sc as plsc

D = 64
L = 16
NC, NS = 2, 16
NW = NC * NS
BLK = 128
EPS = 1e-5
UNROLL = 8
NEWTON = 1


def _ln_impl(idxt, table, gamma, beta):
  nh, nb = idxt.shape
  ntb = nb // BLK
  assert ntb == NW

  mesh = plsc.VectorSubcoreMesh(
      core_axis_name="c", subcore_axis_name="s", num_cores=NC, num_subcores=NS
  )

  @functools.partial(
      pl.kernel,
      out_type=jax.ShapeDtypeStruct((nh, D // 8, ntb, 8, BLK), jnp.float32),
      mesh=mesh,
      compiler_params=pltpu.CompilerParams(
          use_tc_tiling_on_sc=False, needs_layout_passes=False),
      scratch_types=[
          pltpu.VMEM((nh, BLK), jnp.int32),
          pltpu.VMEM((2, BLK, 2 * D), jnp.float32),
          pltpu.VMEM((D // 8, 8, BLK + 1), jnp.float32),
          pltpu.VMEM((D // 8, 8, BLK + 1), jnp.float32),
          pltpu.VMEM((D,), jnp.float32),
          pltpu.VMEM((D,), jnp.float32),
          pltpu.SemaphoreType.DMA,
          pltpu.SemaphoreType.DMA,
          pltpu.SemaphoreType.DMA,
          pltpu.SemaphoreType.DMA,
      ],
  )
  def k(idx_hbm, table_hbm, gamma_hbm, beta_hbm, out_hbm, idxall, ibuf,
        obuf0, obuf1, gamma_v, beta_v, gsem0, gsem1, osem0, osem1):
    obufs = [obuf0, obuf1]
    wid = lax.axis_index("s") * NC + lax.axis_index("c")

    pltpu.sync_copy(gamma_hbm, gamma_v)
    pltpu.sync_copy(beta_hbm, beta_v)
    pltpu.sync_copy(idx_hbm.at[:, pl.ds(wid * BLK, BLK)], idxall)

    gvec = [gamma_v[pl.ds(L * t, L)] for t in range(D // L)]
    bvec = [beta_v[pl.ds(L * t, L)] for t in range(D // L)]
    lane = lax.iota(jnp.int32, L)
    perms = [(lane + sh) & (L - 1) for sh in (8, 4, 2, 1)]
    dvals = [lane + L * t for t in range(D // L)]
    tdvec = [lax.shift_right_logical(dv, 3) for dv in dvals]
    d8vec = [dv & 7 for dv in dvals]

    def fire_gather(h, p, gsem):
      pltpu.async_copy(table_hbm.at[idxall.at[h]], ibuf.at[p], gsem)

    def wait_gather(p, gsem):
      pltpu.make_async_copy(table_hbm.at[idxall.at[0]], ibuf.at[p],
                            gsem).wait()

    def compute(p):
      obuf = obufs[p]

      def rowu(rr, carry):
        for u in range(UNROLL):
          r = rr * UNROLL + u
          x = [ibuf[p, r, pl.ds(L * t, L)] for t in range(D // L)]
          s = (x[0] + x[1]) + (x[2] + x[3])
          sq = (x[0] * x[0] + x[1] * x[1]) + (x[2] * x[2] + x[3] * x[3])
          for pm in perms:
            s = s + s.at[pm].get(mode="promise_in_bounds")
            sq = sq + sq.at[pm].get(mode="promise_in_bounds")
          mean_v = s * (1.0 / D)
          ex2 = sq * (1.0 / D)
          tv = ex2 - mean_v * mean_v + EPS
          seed = lax.bitcast_convert_type(tv, jnp.int32)
          seed = 0x5F3759DF - lax.shift_right_logical(seed, 1)
          g = lax.bitcast_convert_type(seed, jnp.float32)
          htv = 0.5 * tv
          for _ in range(NEWTON):
            g = g * (1.5 - htv * g * g)
          rfull = jnp.full((L,), r, jnp.int32)
          for t in range(D // L):
            y = (x[t] - mean_v) * g * gvec[t] + bvec[t]
            plsc.store_scatter(obuf, [tdvec[t], d8vec[t], rfull], y)
        return carry

      lax.fori_loop(0, BLK // UNROLL, rowu, 0)

    def fire_out(h, p, osem):
      pltpu.async_copy(obufs[p].at[:, :, pl.ds(0, BLK)], out_hbm.at[h, :, wid],
                       osem)

    def wait_out(p, osem):
      pltpu.make_async_copy(obufs[p].at[:, :, pl.ds(0, BLK)],
                            out_hbm.at[0, :, wid], osem).wait()

    fire_gather(0, 0, gsem0)

    def pair_body(i, carry):
      ha = 2 * i
      fire_gather(ha + 1, 1, gsem1)
      wait_gather(0, gsem0)

      @pl.when(i > 0)
      def _():
        wait_out(0, osem0)

      compute(0)
      fire_out(ha, 0, osem0)

      @pl.when(i < nh // 2 - 1)
      def _():
        fire_gather(ha + 2, 0, gsem0)

      wait_gather(1, gsem1)

      @pl.when(i > 0)
      def _():
        wait_out(1, osem1)

      compute(1)
      fire_out(ha + 1, 1, osem1)
      return carry

    lax.fori_loop(0, nh // 2, pair_body, 0)
    wait_out(0, osem0)
    wait_out(1, osem1)

  return k(idxt, table, gamma, beta)


def kernel(job_id, table, gamma, beta):
  b, h = job_id.shape
  idxt = job_id.T.astype(jnp.int32)
  table_p = jnp.pad(table, ((0, 0), (0, D)))
  out5 = _ln_impl(idxt, table_p, gamma, beta)
  out = out5.transpose(2, 4, 0, 1, 3).reshape(b, h, D)
  return out


# speedup vs baseline: 1.3624x; 1.0763x over previous
import functools

import jax
import jax.numpy as jnp
from jax import lax
from jax.experimental import pallas as pl
from jax.experimental.pallas import tpu as pltpu
from jax.experimental.pallas import tpu_sc as plsc

D = 64
L = 16
NC, NS = 2, 16
NW = NC * NS
BLK = 128
K = 2
UNROLL = 4
NEWTON = 1
EPS = 1e-5


def _ln_impl(idx, table, gamma, beta):
  nblk = idx.shape[0]
  blk_per_w = nblk // NW
  nchunk = blk_per_w // K
  npair = nchunk // 2

  mesh = plsc.VectorSubcoreMesh(
      core_axis_name="c", subcore_axis_name="s", num_cores=NC, num_subcores=NS
  )

  @functools.partial(
      pl.kernel,
      out_type=jax.ShapeDtypeStruct((nblk, BLK, D), jnp.float32),
      mesh=mesh,
      compiler_params=pltpu.CompilerParams(use_tc_tiling_on_sc=False),
      scratch_types=[
          pltpu.VMEM((K, BLK), jnp.int32),
          pltpu.VMEM((K, BLK), jnp.int32),
          pltpu.VMEM((K, BLK, D), jnp.float32),
          pltpu.VMEM((K, BLK, D), jnp.float32),
          pltpu.VMEM((K, BLK, D), jnp.float32),
          pltpu.VMEM((K, BLK, D), jnp.float32),
          pltpu.VMEM((D,), jnp.float32),
          pltpu.VMEM((D,), jnp.float32),
          pltpu.SemaphoreType.DMA,
          pltpu.SemaphoreType.DMA,
          pltpu.SemaphoreType.DMA,
          pltpu.SemaphoreType.DMA,
      ],
  )
  def k(idx_hbm, table_hbm, gamma_hbm, beta_hbm, out_hbm, idxv0, idxv1,
        ibuf0, ibuf1, obuf0, obuf1, gamma_v, beta_v, gsem0, gsem1, osem0,
        osem1):
    wid = lax.axis_index("s") * NC + lax.axis_index("c")
    wbase = wid * blk_per_w

    pltpu.sync_copy(gamma_hbm, gamma_v)
    pltpu.sync_copy(beta_hbm, beta_v)
    gvec = [gamma_v[pl.ds(L * t, L)] for t in range(D // L)]
    bvec = [beta_v[pl.ds(L * t, L)] for t in range(D // L)]
    lane = lax.iota(jnp.int32, L)
    perms = [(lane + sh) & (L - 1) for sh in (8, 4, 2, 1)]

    def fire_gathers(blk0, idxv, ibuf, gsem):
      pltpu.sync_copy(idx_hbm.at[pl.ds(blk0, K)], idxv)
      for j in range(K):
        pltpu.async_copy(table_hbm.at[idxv.at[j]], ibuf.at[j], gsem)

    def wait_gathers(ibuf, gsem):
      for j in range(K):
        pltpu.make_async_copy(table_hbm.at[idxv0.at[j]], ibuf.at[j],
                              gsem).wait()

    def row4(ibuf, obuf, j, rr):
      for u in range(UNROLL):
        r = rr * UNROLL + u
        x = [ibuf[j, r, pl.ds(L * t, L)] for t in range(D // L)]
        s = (x[0] + x[1]) + (x[2] + x[3])
        sq = (x[0] * x[0] + x[1] * x[1]) + (x[2] * x[2] + x[3] * x[3])
        for p in perms:
          s = s + s.at[p].get(mode="promise_in_bounds")
          sq = sq + sq.at[p].get(mode="promise_in_bounds")
        mean_v = s * (1.0 / D)
        ex2 = sq * (1.0 / D)
        tv = ex2 - mean_v * mean_v + EPS
        seed = lax.bitcast_convert_type(tv, jnp.int32)
        seed = 0x5F3759DF - lax.shift_right_logical(seed, 1)
        g = lax.bitcast_convert_type(seed, jnp.float32)
        htv = 0.5 * tv
        for _ in range(NEWTON):
          g = g * (1.5 - htv * g * g)
        for t in range(D // L):
          obuf[j, r, pl.ds(L * t, L)] = (x[t] - mean_v) * g * gvec[t] + bvec[t]

    def compute(ibuf, obuf):
      for j in range(K):

        def blk_body(rr, carry, j=j):
          row4(ibuf, obuf, j, rr)
          return carry

        lax.fori_loop(0, BLK // UNROLL, blk_body, 0)

    def fire_out(blk0, obuf, osem):
      pltpu.async_copy(obuf, out_hbm.at[pl.ds(blk0, K)], osem)

    def wait_out(obuf, osem):
      pltpu.make_async_copy(obuf, out_hbm.at[pl.ds(0, K)], osem).wait()

    fire_gathers(wbase, idxv0, ibuf0, gsem0)

    def pair_body(i, carry):
      blk_a = wbase + (2 * i) * K
      blk_b = blk_a + K
      fire_gathers(blk_b, idxv1, ibuf1, gsem1)
      wait_gathers(ibuf0, gsem0)

      @pl.when(i > 0)
      def _():
        wait_out(obuf0, osem0)

      compute(ibuf0, obuf0)
      fire_out(blk_a, obuf0, osem0)

      @pl.when(i < npair - 1)
      def _():
        fire_gathers(blk_b + K, idxv0, ibuf0, gsem0)

      wait_gathers(ibuf1, gsem1)

      @pl.when(i > 0)
      def _():
        wait_out(obuf1, osem1)

      compute(ibuf1, obuf1)
      fire_out(blk_b, obuf1, osem1)
      return carry

    lax.fori_loop(0, npair, pair_body, 0)
    wait_out(obuf0, osem0)
    wait_out(obuf1, osem1)

  return k(idx, table, gamma, beta)


def kernel(job_id, table, gamma, beta):
  b, h = job_id.shape
  n = b * h
  assert n % (NW * BLK * K * 2) == 0
  idx = job_id.reshape(n // BLK, BLK).astype(jnp.int32)
  out = _ln_impl(idx, table, gamma, beta)
  return out.reshape(b, h, D)
